# baseline (speedup 1.0000x reference)
import jax
import jax.numpy as jnp
from jax.experimental import pallas as pl
from jax.experimental.pallas import tpu as pltpu

_H = 2048
_IN = 512
_OUT = 10000
_B = 4096
_SLOPE = 0.01
_OUTP = 10240
_BB1 = 512
_BB2 = 256
_OB = 2048


def _leaky(h):
    return jnp.where(h >= 0, h, _SLOPE * h)


def _dot_t(a, b):
    return jax.lax.dot_general(
        a, b, (((1,), (1,)), ((), ())), preferred_element_type=jnp.float32
    )


def _inv_norm(v):
    vf = v.astype(jnp.float32)
    ss = jnp.sum(vf * vf, axis=1)
    return jax.lax.rsqrt(jnp.maximum(ss, 1e-30))


def _mlp3_body(x_ref, v0_ref, g0_ref, b0_ref, v1_ref, g1_ref, b1_ref,
               v2_ref, g2_ref, b2_ref, out_ref, s_ref):
    i = pl.program_id(0)

    @pl.when(i == 0)
    def _init():
        s_ref[0, :] = g0_ref[0, :] * _inv_norm(v0_ref[...])
        s_ref[1, :] = g1_ref[0, :] * _inv_norm(v1_ref[...])
        s_ref[2, :] = g2_ref[0, :] * _inv_norm(v2_ref[...])

    h = x_ref[...]
    a = _dot_t(h, v0_ref[...])
    h = _leaky(a * s_ref[0:1, :] + b0_ref[...]).astype(jnp.bfloat16)
    a = _dot_t(h, v1_ref[...])
    h = _leaky(a * s_ref[1:2, :] + b1_ref[...]).astype(jnp.bfloat16)
    a = _dot_t(h, v2_ref[...])
    out_ref[...] = _leaky(a * s_ref[2:3, :] + b2_ref[...]).astype(jnp.bfloat16)


def _s3_body(v3_ref, g3_ref, s3_ref):
    s3_ref[0, :] = g3_ref[0, :] * _inv_norm(v3_ref[...])


def _out_body(h_ref, v3_ref, s3_ref, b3_ref, out_ref):
    j = pl.program_id(1)
    a = _dot_t(h_ref[...], v3_ref[...])
    out_ref[:, pl.ds(j * _OB, _OB)] = a * s3_ref[...] + b3_ref[...]

    @pl.when(j == _OUTP // _OB - 1)
    def _softmax():
        full = out_ref[...]
        m = jnp.max(full, axis=1, keepdims=True)
        lse = m + jnp.log(jnp.sum(jnp.exp(full - m), axis=1, keepdims=True))
        out_ref[...] = full - lse


def kernel(x, v0, g0, b0, v1, g1, b1, v2, g2, b2, v3, g3, b3):
    xb = x.astype(jnp.bfloat16)
    v0b = v0.astype(jnp.bfloat16)
    v1b = v1.astype(jnp.bfloat16)
    v2b = v2.astype(jnp.bfloat16)
    npad = _OUTP - _OUT
    v3b = jnp.pad(v3, ((0, npad), (0, 0))).astype(jnp.bfloat16)
    g3p = jnp.pad(g3, (0, npad)).reshape(1, _OUTP)
    b3p = jnp.concatenate(
        [b3, jnp.full((npad,), -jnp.inf, b3.dtype)]).reshape(1, _OUTP)
    g0r, b0r = g0.reshape(1, _H), b0.reshape(1, _H)
    g1r, b1r = g1.reshape(1, _H), b1.reshape(1, _H)
    g2r, b2r = g2.reshape(1, _H), b2.reshape(1, _H)

    full = lambda shape: pl.BlockSpec(shape, lambda i: (0,) * len(shape))
    h3 = pl.pallas_call(
        _mlp3_body,
        grid=(_B // _BB1,),
        in_specs=[
            pl.BlockSpec((_BB1, _IN), lambda i: (i, 0)),
            full((_H, _IN)), full((1, _H)), full((1, _H)),
            full((_H, _H)), full((1, _H)), full((1, _H)),
            full((_H, _H)), full((1, _H)), full((1, _H)),
        ],
        out_specs=pl.BlockSpec((_BB1, _H), lambda i: (i, 0)),
        out_shape=jax.ShapeDtypeStruct((_B, _H), jnp.bfloat16),
        scratch_shapes=[pltpu.VMEM((8, _H), jnp.float32)],
    )(xb, v0b, g0r, b0r, v1b, g1r, b1r, v2b, g2r, b2r)

    s3 = pl.pallas_call(
        _s3_body,
        grid=(_OUTP // _OB,),
        in_specs=[
            pl.BlockSpec((_OB, _H), lambda j: (j, 0)),
            pl.BlockSpec((1, _OB), lambda j: (0, j)),
        ],
        out_specs=pl.BlockSpec((1, _OB), lambda j: (0, j)),
        out_shape=jax.ShapeDtypeStruct((1, _OUTP), jnp.float32),
    )(v3b, g3p)

    outp = pl.pallas_call(
        _out_body,
        grid=(_B // _BB2, _OUTP // _OB),
        in_specs=[
            pl.BlockSpec((_BB2, _H), lambda i, j: (i, 0)),
            pl.BlockSpec((_OB, _H), lambda i, j: (j, 0)),
            pl.BlockSpec((1, _OB), lambda i, j: (0, j)),
            pl.BlockSpec((1, _OB), lambda i, j: (0, j)),
        ],
        out_specs=pl.BlockSpec((_BB2, _OUTP), lambda i, j: (i, 0)),
        out_shape=jax.ShapeDtypeStruct((_B, _OUTP), jnp.float32),
    )(h3, v3b, s3, b3p)

    return outp[:, :_OUT]


# speedup vs baseline: 1.1190x; 1.1190x over previous
import jax
import jax.numpy as jnp
from jax.experimental import pallas as pl
from jax.experimental.pallas import tpu as pltpu

_H = 2048
_IN = 512
_OUT = 10000
_B = 4096
_SLOPE = 0.01
_BB1 = 512
_BB2 = 256
_OB = 2048
_NJ = 5
_TAIL = _OUT - (_NJ - 1) * _OB


def _leaky(y):
    return jnp.maximum(y, _SLOPE * y)


def _dot(a, b):
    return jax.lax.dot_general(
        a, b, (((1,), (0,)), ((), ())), preferred_element_type=jnp.float32
    )


def _col_scale(vt, g):
    vf = vt[...].astype(jnp.float32)
    ss = jnp.sum(vf * vf, axis=0, keepdims=True)
    return g * jax.lax.rsqrt(jnp.maximum(ss, 1e-30))


def _mlp3_body(x_ref, v0_ref, g0_ref, b0_ref, v1_ref, g1_ref, b1_ref,
               v2_ref, g2_ref, b2_ref, out_ref, s_ref):
    i = pl.program_id(0)

    @pl.when(i == 0)
    def _init():
        s_ref[0:1, :] = _col_scale(v0_ref, g0_ref[...])
        s_ref[1:2, :] = _col_scale(v1_ref, g1_ref[...])
        s_ref[2:3, :] = _col_scale(v2_ref, g2_ref[...])

    h = x_ref[...].astype(jnp.bfloat16)
    a = _dot(h, v0_ref[...])
    h = _leaky(a * s_ref[0:1, :] + b0_ref[...]).astype(jnp.bfloat16)
    a = _dot(h, v1_ref[...])
    h = _leaky(a * s_ref[1:2, :] + b1_ref[...]).astype(jnp.bfloat16)
    a = _dot(h, v2_ref[...])
    out_ref[...] = _leaky(a * s_ref[2:3, :] + b2_ref[...]).astype(jnp.bfloat16)


def _s3_body(v3_ref, g3_ref, s3_ref):
    s3_ref[...] = _col_scale(v3_ref, g3_ref[...])


def _out_body(h_ref, v3_ref, s3_ref, b3_ref, out_ref, m_ref, l_ref):
    j = pl.program_id(1)
    t = _dot(h_ref[...], v3_ref[...]) * s3_ref[...] + b3_ref[...]

    @pl.when(j == 0)
    def _reset():
        m_ref[...] = jnp.full_like(m_ref, -jnp.inf)
        l_ref[...] = jnp.zeros_like(l_ref)

    @pl.when(j < _NJ - 1)
    def _store_full():
        out_ref[:, pl.ds(j * _OB, _OB)] = t

    @pl.when(j == _NJ - 1)
    def _store_tail():
        out_ref[:, (_NJ - 1) * _OB:_OUT] = t[:, :_TAIL]

    lane = jax.lax.broadcasted_iota(jnp.int32, t.shape, 1)
    valid = jnp.where(j == _NJ - 1, _TAIL, _OB)
    tv = jnp.where(lane < valid, t, -jnp.inf)
    m_old = m_ref[:, 0:1]
    m_new = jnp.maximum(m_old, jnp.max(tv, axis=1, keepdims=True))
    l_ref[:, 0:1] = (l_ref[:, 0:1] * jnp.exp(m_old - m_new)
                     + jnp.sum(jnp.exp(tv - m_new), axis=1, keepdims=True))
    m_ref[:, 0:1] = m_new

    @pl.when(j == _NJ - 1)
    def _finish():
        lse = m_ref[:, 0:1] + jnp.log(l_ref[:, 0:1])
        out_ref[...] = out_ref[...] - lse


def kernel(x, v0, g0, b0, v1, g1, b1, v2, g2, b2, v3, g3, b3):
    bf16 = jnp.bfloat16
    v0t = v0.T.astype(bf16)
    v1t = v1.T.astype(bf16)
    v2t = v2.T.astype(bf16)
    v3t = v3.T.astype(bf16)
    g0r, b0r = g0.reshape(1, _H), b0.reshape(1, _H)
    g1r, b1r = g1.reshape(1, _H), b1.reshape(1, _H)
    g2r, b2r = g2.reshape(1, _H), b2.reshape(1, _H)
    g3r, b3r = g3.reshape(1, _OUT), b3.reshape(1, _OUT)

    full = lambda shape: pl.BlockSpec(shape, lambda i: (0,) * len(shape))
    h3 = pl.pallas_call(
        _mlp3_body,
        grid=(_B // _BB1,),
        in_specs=[
            pl.BlockSpec((_BB1, _IN), lambda i: (i, 0)),
            full((_IN, _H)), full((1, _H)), full((1, _H)),
            full((_H, _H)), full((1, _H)), full((1, _H)),
            full((_H, _H)), full((1, _H)), full((1, _H)),
        ],
        out_specs=pl.BlockSpec((_BB1, _H), lambda i: (i, 0)),
        out_shape=jax.ShapeDtypeStruct((_B, _H), bf16),
        scratch_shapes=[pltpu.VMEM((8, _H), jnp.float32)],
    )(x, v0t, g0r, b0r, v1t, g1r, b1r, v2t, g2r, b2r)

    s3 = pl.pallas_call(
        _s3_body,
        grid=(_NJ,),
        in_specs=[
            pl.BlockSpec((_H, _OB), lambda j: (0, j)),
            pl.BlockSpec((1, _OB), lambda j: (0, j)),
        ],
        out_specs=pl.BlockSpec((1, _OB), lambda j: (0, j)),
        out_shape=jax.ShapeDtypeStruct((1, _OUT), jnp.float32),
    )(v3t, g3r)

    out = pl.pallas_call(
        _out_body,
        grid=(_B // _BB2, _NJ),
        in_specs=[
            pl.BlockSpec((_BB2, _H), lambda i, j: (i, 0)),
            pl.BlockSpec((_H, _OB), lambda i, j: (0, j)),
            pl.BlockSpec((1, _OB), lambda i, j: (0, j)),
            pl.BlockSpec((1, _OB), lambda i, j: (0, j)),
        ],
        out_specs=pl.BlockSpec((_BB2, _OUT), lambda i, j: (i, 0)),
        out_shape=jax.ShapeDtypeStruct((_B, _OUT), jnp.float32),
        scratch_shapes=[
            pltpu.VMEM((_BB2, 128), jnp.float32),
            pltpu.VMEM((_BB2, 128), jnp.float32),
        ],
    )(h3, v3t, s3, b3r)

    return out


# speedup vs baseline: 1.1785x; 1.0532x over previous
import jax
import jax.numpy as jnp
from jax.experimental import pallas as pl
from jax.experimental.pallas import tpu as pltpu

_H = 2048
_IN = 512
_OUT = 10000
_B = 4096
_SLOPE = 0.01
_BB1 = 512
_BB2 = 256
_OB = 2048
_NJ = 5
_TAIL = _OUT - (_NJ - 1) * _OB
_OUTP = _NJ * _OB


def _leaky(y):
    return jnp.maximum(y, _SLOPE * y)


def _dot(a, b):
    return jax.lax.dot_general(
        a, b, (((1,), (0,)), ((), ())), preferred_element_type=jnp.float32
    )


def _prep_body(v_ref, g_ref, vt_ref, s_ref):
    vt = v_ref[...].T
    vt_ref[...] = vt.astype(jnp.bfloat16)
    ss = jnp.sum(vt * vt, axis=0, keepdims=True)
    s_ref[...] = g_ref[...] * jax.lax.rsqrt(jnp.maximum(ss, 1e-30))


def _prep(v, g, n, k):
    return pl.pallas_call(
        _prep_body,
        grid=(1,),
        in_specs=[
            pl.BlockSpec((n, k), lambda i: (0, 0)),
            pl.BlockSpec((1, n), lambda i: (0, 0)),
        ],
        out_specs=[
            pl.BlockSpec((k, n), lambda i: (0, 0)),
            pl.BlockSpec((1, n), lambda i: (0, 0)),
        ],
        out_shape=[
            jax.ShapeDtypeStruct((k, n), jnp.bfloat16),
            jax.ShapeDtypeStruct((1, n), jnp.float32),
        ],
    )(v, g.reshape(1, n))


def _prep3(v3, g3):
    return pl.pallas_call(
        _prep_body,
        grid=(_NJ,),
        in_specs=[
            pl.BlockSpec((_OB, _H), lambda j: (j, 0)),
            pl.BlockSpec((1, _OB), lambda j: (0, j)),
        ],
        out_specs=[
            pl.BlockSpec((_H, _OB), lambda j: (0, j)),
            pl.BlockSpec((1, _OB), lambda j: (0, j)),
        ],
        out_shape=[
            jax.ShapeDtypeStruct((_H, _OUTP), jnp.bfloat16),
            jax.ShapeDtypeStruct((1, _OUTP), jnp.float32),
        ],
    )(v3, jnp.pad(g3, (0, _OUTP - _OUT)).reshape(1, _OUTP))


def _mlp3_body(x_ref, v0_ref, s0_ref, b0_ref, v1_ref, s1_ref, b1_ref,
               v2_ref, s2_ref, b2_ref, out_ref):
    h = x_ref[...].astype(jnp.bfloat16)
    a = _dot(h, v0_ref[...])
    h = _leaky(a * s0_ref[...] + b0_ref[...]).astype(jnp.bfloat16)
    a = _dot(h, v1_ref[...])
    h = _leaky(a * s1_ref[...] + b1_ref[...]).astype(jnp.bfloat16)
    a = _dot(h, v2_ref[...])
    out_ref[...] = _leaky(a * s2_ref[...] + b2_ref[...]).astype(jnp.bfloat16)


def _out_body(h_ref, v3_ref, s3_ref, b3_ref, out_ref, m_ref, l_ref):
    j = pl.program_id(1)
    t = _dot(h_ref[...], v3_ref[...]) * s3_ref[...] + b3_ref[...]

    @pl.when(j == 0)
    def _reset():
        m_ref[...] = jnp.full_like(m_ref, -jnp.inf)
        l_ref[...] = jnp.zeros_like(l_ref)

    @pl.when(j < _NJ - 1)
    def _store_full():
        out_ref[:, pl.ds(j * _OB, _OB)] = t

    @pl.when(j == _NJ - 1)
    def _store_tail():
        out_ref[:, (_NJ - 1) * _OB:_OUT] = t[:, :_TAIL]

    lane = jax.lax.broadcasted_iota(jnp.int32, t.shape, 1)
    valid = jnp.where(j == _NJ - 1, _TAIL, _OB)
    tv = jnp.where(lane < valid, t, -jnp.inf)
    m_old = m_ref[:, 0:1]
    m_new = jnp.maximum(m_old, jnp.max(tv, axis=1, keepdims=True))
    l_ref[:, 0:1] = (l_ref[:, 0:1] * jnp.exp(m_old - m_new)
                     + jnp.sum(jnp.exp(tv - m_new), axis=1, keepdims=True))
    m_ref[:, 0:1] = m_new

    @pl.when(j == _NJ - 1)
    def _finish():
        lse = m_ref[:, 0:1] + jnp.log(l_ref[:, 0:1])
        out_ref[...] = out_ref[...] - lse


def kernel(x, v0, g0, b0, v1, g1, b1, v2, g2, b2, v3, g3, b3):
    v0t, s0 = _prep(v0, g0, _H, _IN)
    v1t, s1 = _prep(v1, g1, _H, _H)
    v2t, s2 = _prep(v2, g2, _H, _H)
    v3t, s3 = _prep3(v3, g3)
    b0r, b1r, b2r = b0.reshape(1, _H), b1.reshape(1, _H), b2.reshape(1, _H)
    b3r = b3.reshape(1, _OUT)

    full = lambda shape: pl.BlockSpec(shape, lambda i: (0,) * len(shape))
    h3 = pl.pallas_call(
        _mlp3_body,
        grid=(_B // _BB1,),
        in_specs=[
            pl.BlockSpec((_BB1, _IN), lambda i: (i, 0)),
            full((_IN, _H)), full((1, _H)), full((1, _H)),
            full((_H, _H)), full((1, _H)), full((1, _H)),
            full((_H, _H)), full((1, _H)), full((1, _H)),
        ],
        out_specs=pl.BlockSpec((_BB1, _H), lambda i: (i, 0)),
        out_shape=jax.ShapeDtypeStruct((_B, _H), jnp.bfloat16),
    )(x, v0t, s0, b0r, v1t, s1, b1r, v2t, s2, b2r)

    out = pl.pallas_call(
        _out_body,
        grid=(_B // _BB2, _NJ),
        in_specs=[
            pl.BlockSpec((_BB2, _H), lambda i, j: (i, 0)),
            pl.BlockSpec((_H, _OB), lambda i, j: (0, j)),
            pl.BlockSpec((1, _OB), lambda i, j: (0, j)),
            pl.BlockSpec((1, _OB), lambda i, j: (0, j)),
        ],
        out_specs=pl.BlockSpec((_BB2, _OUT), lambda i, j: (i, 0)),
        out_shape=jax.ShapeDtypeStruct((_B, _OUT), jnp.float32),
        scratch_shapes=[
            pltpu.VMEM((_BB2, 128), jnp.float32),
            pltpu.VMEM((_BB2, 128), jnp.float32),
        ],
    )(h3, v3t, s3, b3r)

    return out
